# baseline (speedup 1.0000x reference)
import jax
import jax.numpy as jnp
from jax import lax
from jax.experimental import pallas as pl
from jax.experimental.pallas import tpu as pltpu
from jax.experimental.pallas import tpu_sc as plsc

EMB_ROWS = 1000000
EMB_DIM = 32
BATCH = 16384
SC_BATCH = 8192
TC_BATCH = BATCH - SC_BATCH
NUM_CORES = 2
NUM_SUBCORES = 16
LANES = 16
NUM_WORKERS = NUM_CORES * NUM_SUBCORES
ROWS_PER_WORKER = SC_BATCH // NUM_WORKERS
CHUNK = 128
NCHUNKS = ROWS_PER_WORKER // CHUNK
GROUPS = CHUNK // LANES


def _sc_body(p1_hbm, p2_hbm, table_hbm, out_hbm,
             idx1_v, idx2_v, rows1_v, rows2_v, out_v,
             sem1a, sem1b, sem2a, sem2b):
    wid = lax.axis_index("s") * NUM_CORES + lax.axis_index("c")
    base = wid * ROWS_PER_WORKER

    pltpu.sync_copy(p1_hbm.at[pl.ds(base, ROWS_PER_WORKER)], idx1_v)
    pltpu.sync_copy(p2_hbm.at[pl.ds(base, ROWS_PER_WORKER)], idx2_v)

    sems1 = (sem1a, sem1b)
    sems2 = (sem2a, sem2b)

    def start_chunk(c, buf):
        def issue(g, carry):
            iv1 = idx1_v[pl.ds(c * CHUNK + g * LANES, LANES)]
            iv2 = idx2_v[pl.ds(c * CHUNK + g * LANES, LANES)]
            for r in range(LANES):
                slot = g * LANES + r
                pltpu.async_copy(table_hbm.at[pl.ds(iv1[r], 1)],
                                 rows1_v.at[buf, pl.ds(slot, 1)], sems1[buf])
                pltpu.async_copy(table_hbm.at[pl.ds(iv2[r], 1)],
                                 rows2_v.at[buf, pl.ds(slot, 1)], sems2[buf])
            return carry
        lax.fori_loop(0, GROUPS, issue, 0)

    def wait_chunk(buf):
        def drain(r, carry):
            pltpu.make_async_copy(table_hbm.at[pl.ds(0, 1)],
                                  rows1_v.at[buf, pl.ds(0, 1)],
                                  sems1[buf]).wait()
            pltpu.make_async_copy(table_hbm.at[pl.ds(0, 1)],
                                  rows2_v.at[buf, pl.ds(0, 1)],
                                  sems2[buf]).wait()
            return carry
        lax.fori_loop(0, CHUNK, drain, 0)

    lane = lax.iota(jnp.int32, LANES)

    def shuffle(v, perm):
        return lax.gather(
            v, perm[:, None],
            lax.GatherDimensionNumbers(
                offset_dims=(), collapsed_slice_dims=(0,),
                start_index_map=(0,)),
            slice_sizes=(1,),
            mode=lax.GatherScatterMode.PROMISE_IN_BOUNDS)

    def combine(a, b, k):
        m = (lane & k) == 0
        sel_ab = jnp.where(m, a, b)
        sel_ba = jnp.where(m, b, a)
        return sel_ab + shuffle(sel_ba, lane ^ k)

    def compute_chunk(buf, out_base):
        r1 = rows1_v.at[buf]
        r2 = rows2_v.at[buf]

        def group(g, carry):
            row0 = g * LANES
            w = []
            for r in range(LANES):
                row = row0 + r
                a0 = r1[row, pl.ds(0, LANES)]
                a1 = r1[row, pl.ds(LANES, LANES)]
                b0 = r2[row, pl.ds(0, LANES)]
                b1 = r2[row, pl.ds(LANES, LANES)]
                w.append(a0 * b0 + a1 * b1)
            for k in (1, 2, 4, 8):
                w = [combine(w[2 * i], w[2 * i + 1], k)
                     for i in range(len(w) // 2)]
            acc = w[0]
            out_v[pl.ds(out_base + row0, LANES)] = 1.0 / (1.0 + jnp.exp(-acc))
            return carry

        lax.fori_loop(0, GROUPS, group, 0)

    start_chunk(0, 0)
    for c in range(NCHUNKS):
        buf = c % 2
        if c + 1 < NCHUNKS:
            start_chunk(c + 1, 1 - buf)
        wait_chunk(buf)
        compute_chunk(buf, c * CHUNK)

    pltpu.sync_copy(out_v, out_hbm.at[pl.ds(base, ROWS_PER_WORKER)])


def _sc_half(p1, p2, table):
    mesh = plsc.VectorSubcoreMesh(core_axis_name="c", subcore_axis_name="s")
    run = pl.kernel(
        _sc_body,
        mesh=mesh,
        out_type=jax.ShapeDtypeStruct((SC_BATCH,), jnp.float32),
        scratch_types=[
            pltpu.VMEM((ROWS_PER_WORKER,), jnp.int32),
            pltpu.VMEM((ROWS_PER_WORKER,), jnp.int32),
            pltpu.VMEM((2, CHUNK, EMB_DIM), jnp.float32),
            pltpu.VMEM((2, CHUNK, EMB_DIM), jnp.float32),
            pltpu.VMEM((ROWS_PER_WORKER,), jnp.float32),
            pltpu.SemaphoreType.DMA,
            pltpu.SemaphoreType.DMA,
            pltpu.SemaphoreType.DMA,
            pltpu.SemaphoreType.DMA,
        ],
    )
    return run(p1, p2, table)


def _tc_body(p1_s, p2_s, table_hbm, out_v, rows1_v, rows2_v, sem1, sem2):
    def issue(r, carry):
        i1 = p1_s[r]
        i2 = p2_s[r]
        pltpu.make_async_copy(table_hbm.at[pl.ds(i1, 1)],
                              rows1_v.at[pl.ds(r, 1)], sem1).start()
        pltpu.make_async_copy(table_hbm.at[pl.ds(i2, 1)],
                              rows2_v.at[pl.ds(r, 1)], sem2).start()
        return carry

    lax.fori_loop(0, TC_BATCH, issue, 0)

    def drain(r, carry):
        pltpu.make_async_copy(table_hbm.at[pl.ds(0, 1)],
                              rows1_v.at[pl.ds(0, 1)], sem1).wait()
        pltpu.make_async_copy(table_hbm.at[pl.ds(0, 1)],
                              rows2_v.at[pl.ds(0, 1)], sem2).wait()
        return carry

    lax.fori_loop(0, TC_BATCH, drain, 0)

    prod = rows1_v[...] * rows2_v[...]
    s = jnp.sum(prod, axis=1)
    out_v[...] = 1.0 / (1.0 + jnp.exp(-s))


def _tc_half(p1, p2, table):
    return pl.pallas_call(
        _tc_body,
        out_shape=jax.ShapeDtypeStruct((TC_BATCH,), jnp.float32),
        in_specs=[
            pl.BlockSpec(memory_space=pltpu.SMEM),
            pl.BlockSpec(memory_space=pltpu.SMEM),
            pl.BlockSpec(memory_space=pl.ANY),
        ],
        out_specs=pl.BlockSpec(memory_space=pltpu.VMEM),
        scratch_shapes=[
            pltpu.VMEM((TC_BATCH, EMB_DIM), jnp.float32),
            pltpu.VMEM((TC_BATCH, EMB_DIM), jnp.float32),
            pltpu.SemaphoreType.DMA,
            pltpu.SemaphoreType.DMA,
        ],
    )(p1, p2, table)


def kernel(product1, product2, embedding_weight):
    p1 = product1.astype(jnp.int32)
    p2 = product2.astype(jnp.int32)
    out_sc = _sc_half(p1[:SC_BATCH], p2[:SC_BATCH], embedding_weight)
    out_tc = _tc_half(p1[SC_BATCH:], p2[SC_BATCH:], embedding_weight)
    return jnp.concatenate([out_sc, out_tc])


# speedup vs baseline: 1.1435x; 1.1435x over previous
import jax
import jax.numpy as jnp
from jax import lax
from jax.experimental import pallas as pl
from jax.experimental.pallas import tpu as pltpu
from jax.experimental.pallas import tpu_sc as plsc

EMB_ROWS = 1000000
EMB_DIM = 32
BATCH = 16384
SC_BATCH = 8192
TC_BATCH = BATCH - SC_BATCH
NUM_CORES = 2
NUM_SUBCORES = 16
LANES = 16
NUM_WORKERS = NUM_CORES * NUM_SUBCORES
ROWS_PER_WORKER = SC_BATCH // NUM_WORKERS
CHUNK = 128
NCHUNKS = ROWS_PER_WORKER // CHUNK
GROUPS = CHUNK // LANES


def _sc_body(p1_hbm, p2_hbm, table_hbm, out_hbm,
             idx1_v, idx2_v, rows1_v, rows2_v, out_v,
             sem1a, sem1b, sem2a, sem2b):
    wid = lax.axis_index("s") * NUM_CORES + lax.axis_index("c")
    base = wid * ROWS_PER_WORKER

    pltpu.sync_copy(p1_hbm.at[pl.ds(base, ROWS_PER_WORKER)], idx1_v)
    pltpu.sync_copy(p2_hbm.at[pl.ds(base, ROWS_PER_WORKER)], idx2_v)

    sems1 = (sem1a, sem1b)
    sems2 = (sem2a, sem2b)

    def start_chunk(c, buf):
        def issue(g, carry):
            iv1 = idx1_v[pl.ds(c * CHUNK + g * LANES, LANES)]
            iv2 = idx2_v[pl.ds(c * CHUNK + g * LANES, LANES)]
            for r in range(LANES):
                slot = g * LANES + r
                pltpu.async_copy(table_hbm.at[pl.ds(iv1[r], 1)],
                                 rows1_v.at[buf, pl.ds(slot, 1)], sems1[buf])
                pltpu.async_copy(table_hbm.at[pl.ds(iv2[r], 1)],
                                 rows2_v.at[buf, pl.ds(slot, 1)], sems2[buf])
            return carry
        lax.fori_loop(0, GROUPS, issue, 0)

    def wait_chunk(buf):
        def drain(r, carry):
            pltpu.make_async_copy(table_hbm.at[pl.ds(0, 1)],
                                  rows1_v.at[buf, pl.ds(0, 1)],
                                  sems1[buf]).wait()
            pltpu.make_async_copy(table_hbm.at[pl.ds(0, 1)],
                                  rows2_v.at[buf, pl.ds(0, 1)],
                                  sems2[buf]).wait()
            return carry
        lax.fori_loop(0, CHUNK, drain, 0)

    lane = lax.iota(jnp.int32, LANES)

    def shuffle(v, perm):
        return lax.gather(
            v, perm[:, None],
            lax.GatherDimensionNumbers(
                offset_dims=(), collapsed_slice_dims=(0,),
                start_index_map=(0,)),
            slice_sizes=(1,),
            mode=lax.GatherScatterMode.PROMISE_IN_BOUNDS)

    def combine(a, b, k):
        m = (lane & k) == 0
        sel_ab = jnp.where(m, a, b)
        sel_ba = jnp.where(m, b, a)
        return sel_ab + shuffle(sel_ba, lane ^ k)

    def compute_chunk(buf, out_base):
        r1 = rows1_v.at[buf]
        r2 = rows2_v.at[buf]

        def group(g, carry):
            row0 = g * LANES
            w = []
            for r in range(LANES):
                row = row0 + r
                a0 = r1[row, pl.ds(0, LANES)]
                a1 = r1[row, pl.ds(LANES, LANES)]
                b0 = r2[row, pl.ds(0, LANES)]
                b1 = r2[row, pl.ds(LANES, LANES)]
                w.append(a0 * b0 + a1 * b1)
            for k in (1, 2, 4, 8):
                w = [combine(w[2 * i], w[2 * i + 1], k)
                     for i in range(len(w) // 2)]
            acc = w[0]
            out_v[pl.ds(out_base + row0, LANES)] = 1.0 / (1.0 + jnp.exp(-acc))
            return carry

        lax.fori_loop(0, GROUPS, group, 0)

    start_chunk(0, 0)
    for c in range(NCHUNKS):
        buf = c % 2
        if c + 1 < NCHUNKS:
            start_chunk(c + 1, 1 - buf)
        wait_chunk(buf)
        compute_chunk(buf, c * CHUNK)

    pltpu.sync_copy(out_v, out_hbm.at[pl.ds(base, ROWS_PER_WORKER)])


def _sc_half(p1, p2, table):
    mesh = plsc.VectorSubcoreMesh(core_axis_name="c", subcore_axis_name="s")
    run = pl.kernel(
        _sc_body,
        mesh=mesh,
        out_type=jax.ShapeDtypeStruct((SC_BATCH,), jnp.float32),
        scratch_types=[
            pltpu.VMEM((ROWS_PER_WORKER,), jnp.int32),
            pltpu.VMEM((ROWS_PER_WORKER,), jnp.int32),
            pltpu.VMEM((2, CHUNK, EMB_DIM), jnp.float32),
            pltpu.VMEM((2, CHUNK, EMB_DIM), jnp.float32),
            pltpu.VMEM((ROWS_PER_WORKER,), jnp.float32),
            pltpu.SemaphoreType.DMA,
            pltpu.SemaphoreType.DMA,
            pltpu.SemaphoreType.DMA,
            pltpu.SemaphoreType.DMA,
        ],
    )
    return run(p1, p2, table)


def _tc_body(p1_s, p2_s, table_hbm, out_v, rows1_v, rows2_v, sem1, sem2):
    def issue(g, carry):
        for u in range(8):
            r = g * 8 + u
            i1 = p1_s[r]
            i2 = p2_s[r]
            pltpu.make_async_copy(table_hbm.at[pl.ds(i1, 1)],
                                  rows1_v.at[pl.ds(r, 1)], sem1).start()
            pltpu.make_async_copy(table_hbm.at[pl.ds(i2, 1)],
                                  rows2_v.at[pl.ds(r, 1)], sem2).start()
        return carry

    lax.fori_loop(0, TC_BATCH // 8, issue, 0)

    pltpu.make_async_copy(table_hbm.at[pl.ds(0, TC_BATCH)],
                          rows1_v, sem1).wait()
    pltpu.make_async_copy(table_hbm.at[pl.ds(0, TC_BATCH)],
                          rows2_v, sem2).wait()

    prod = rows1_v[...] * rows2_v[...]
    s = jnp.sum(prod, axis=1)
    out_v[...] = 1.0 / (1.0 + jnp.exp(-s))


def _tc_half(p1, p2, table):
    return pl.pallas_call(
        _tc_body,
        out_shape=jax.ShapeDtypeStruct((TC_BATCH,), jnp.float32),
        in_specs=[
            pl.BlockSpec(memory_space=pltpu.SMEM),
            pl.BlockSpec(memory_space=pltpu.SMEM),
            pl.BlockSpec(memory_space=pl.ANY),
        ],
        out_specs=pl.BlockSpec(memory_space=pltpu.VMEM),
        scratch_shapes=[
            pltpu.VMEM((TC_BATCH, EMB_DIM), jnp.float32),
            pltpu.VMEM((TC_BATCH, EMB_DIM), jnp.float32),
            pltpu.SemaphoreType.DMA,
            pltpu.SemaphoreType.DMA,
        ],
    )(p1, p2, table)


def kernel(product1, product2, embedding_weight):
    p1 = product1.astype(jnp.int32)
    p2 = product2.astype(jnp.int32)
    out_sc = _sc_half(p1[:SC_BATCH], p2[:SC_BATCH], embedding_weight)
    out_tc = _tc_half(p1[SC_BATCH:], p2[SC_BATCH:], embedding_weight)
    return jnp.concatenate([out_sc, out_tc])


# speedup vs baseline: 1.1870x; 1.0381x over previous
import jax
import jax.numpy as jnp
from jax import lax
from jax.experimental import pallas as pl
from jax.experimental.pallas import tpu as pltpu
from jax.experimental.pallas import tpu_sc as plsc

EMB_ROWS = 1000000
EMB_DIM = 32
BATCH = 16384
SC_BATCH = 10240
TC_BATCH = BATCH - SC_BATCH
NUM_CORES = 2
NUM_SUBCORES = 16
LANES = 16
NUM_WORKERS = NUM_CORES * NUM_SUBCORES
ROWS_PER_WORKER = SC_BATCH // NUM_WORKERS
CHUNK = 160
NCHUNKS = ROWS_PER_WORKER // CHUNK
GROUPS = CHUNK // LANES


def _sc_body(p1_hbm, p2_hbm, table_hbm, out_hbm,
             idx1_v, idx2_v, rows1_v, rows2_v, out_v,
             sem1a, sem1b, sem2a, sem2b):
    wid = lax.axis_index("s") * NUM_CORES + lax.axis_index("c")
    base = wid * ROWS_PER_WORKER

    pltpu.sync_copy(p1_hbm.at[pl.ds(base, ROWS_PER_WORKER)], idx1_v)
    pltpu.sync_copy(p2_hbm.at[pl.ds(base, ROWS_PER_WORKER)], idx2_v)

    sems1 = (sem1a, sem1b)
    sems2 = (sem2a, sem2b)

    def start_chunk(c, buf):
        def issue(g, carry):
            iv1 = idx1_v[pl.ds(c * CHUNK + g * LANES, LANES)]
            iv2 = idx2_v[pl.ds(c * CHUNK + g * LANES, LANES)]
            for r in range(LANES):
                slot = g * LANES + r
                pltpu.async_copy(table_hbm.at[pl.ds(iv1[r], 1)],
                                 rows1_v.at[buf, pl.ds(slot, 1)], sems1[buf])
                pltpu.async_copy(table_hbm.at[pl.ds(iv2[r], 1)],
                                 rows2_v.at[buf, pl.ds(slot, 1)], sems2[buf])
            return carry
        lax.fori_loop(0, GROUPS, issue, 0)

    def wait_chunk(buf):
        def drain(r, carry):
            pltpu.make_async_copy(table_hbm.at[pl.ds(0, 1)],
                                  rows1_v.at[buf, pl.ds(0, 1)],
                                  sems1[buf]).wait()
            pltpu.make_async_copy(table_hbm.at[pl.ds(0, 1)],
                                  rows2_v.at[buf, pl.ds(0, 1)],
                                  sems2[buf]).wait()
            return carry
        lax.fori_loop(0, CHUNK, drain, 0)

    lane = lax.iota(jnp.int32, LANES)

    def shuffle(v, perm):
        return lax.gather(
            v, perm[:, None],
            lax.GatherDimensionNumbers(
                offset_dims=(), collapsed_slice_dims=(0,),
                start_index_map=(0,)),
            slice_sizes=(1,),
            mode=lax.GatherScatterMode.PROMISE_IN_BOUNDS)

    def combine(a, b, k):
        m = (lane & k) == 0
        sel_ab = jnp.where(m, a, b)
        sel_ba = jnp.where(m, b, a)
        return sel_ab + shuffle(sel_ba, lane ^ k)

    def compute_chunk(buf, out_base):
        r1 = rows1_v.at[buf]
        r2 = rows2_v.at[buf]

        def group(g, carry):
            row0 = g * LANES
            w = []
            for r in range(LANES):
                row = row0 + r
                a0 = r1[row, pl.ds(0, LANES)]
                a1 = r1[row, pl.ds(LANES, LANES)]
                b0 = r2[row, pl.ds(0, LANES)]
                b1 = r2[row, pl.ds(LANES, LANES)]
                w.append(a0 * b0 + a1 * b1)
            for k in (1, 2, 4, 8):
                w = [combine(w[2 * i], w[2 * i + 1], k)
                     for i in range(len(w) // 2)]
            acc = w[0]
            out_v[pl.ds(out_base + row0, LANES)] = 1.0 / (1.0 + jnp.exp(-acc))
            return carry

        lax.fori_loop(0, GROUPS, group, 0)

    start_chunk(0, 0)
    for c in range(NCHUNKS):
        buf = c % 2
        if c + 1 < NCHUNKS:
            start_chunk(c + 1, 1 - buf)
        wait_chunk(buf)
        compute_chunk(buf, c * CHUNK)

    pltpu.sync_copy(out_v, out_hbm.at[pl.ds(base, ROWS_PER_WORKER)])


def _sc_half(p1, p2, table):
    mesh = plsc.VectorSubcoreMesh(core_axis_name="c", subcore_axis_name="s")
    run = pl.kernel(
        _sc_body,
        mesh=mesh,
        out_type=jax.ShapeDtypeStruct((SC_BATCH,), jnp.float32),
        scratch_types=[
            pltpu.VMEM((ROWS_PER_WORKER,), jnp.int32),
            pltpu.VMEM((ROWS_PER_WORKER,), jnp.int32),
            pltpu.VMEM((2, CHUNK, EMB_DIM), jnp.float32),
            pltpu.VMEM((2, CHUNK, EMB_DIM), jnp.float32),
            pltpu.VMEM((ROWS_PER_WORKER,), jnp.float32),
            pltpu.SemaphoreType.DMA,
            pltpu.SemaphoreType.DMA,
            pltpu.SemaphoreType.DMA,
            pltpu.SemaphoreType.DMA,
        ],
    )
    return run(p1, p2, table)


def _tc_body(p1_s, p2_s, table_hbm, out_v, rows1_v, rows2_v, *sems):
    def issue(g, carry):
        for u in range(8):
            r = g * 8 + u
            i1 = p1_s[r]
            i2 = p2_s[r]
            pltpu.make_async_copy(table_hbm.at[pl.ds(i1, 1)],
                                  rows1_v.at[pl.ds(r, 1)], sems[u % 4]).start()
            pltpu.make_async_copy(table_hbm.at[pl.ds(i2, 1)],
                                  rows2_v.at[pl.ds(r, 1)], sems[4 + u % 4]).start()
        return carry

    lax.fori_loop(0, TC_BATCH // 8, issue, 0)

    for q in range(4):
        pltpu.make_async_copy(table_hbm.at[pl.ds(0, TC_BATCH // 4)],
                              rows1_v.at[pl.ds(0, TC_BATCH // 4)],
                              sems[q]).wait()
        pltpu.make_async_copy(table_hbm.at[pl.ds(0, TC_BATCH // 4)],
                              rows2_v.at[pl.ds(0, TC_BATCH // 4)],
                              sems[4 + q]).wait()

    prod = rows1_v[...] * rows2_v[...]
    s = jnp.sum(prod, axis=1)
    out_v[...] = 1.0 / (1.0 + jnp.exp(-s))


def _tc_half(p1, p2, table):
    return pl.pallas_call(
        _tc_body,
        out_shape=jax.ShapeDtypeStruct((TC_BATCH,), jnp.float32),
        in_specs=[
            pl.BlockSpec(memory_space=pltpu.SMEM),
            pl.BlockSpec(memory_space=pltpu.SMEM),
            pl.BlockSpec(memory_space=pl.ANY),
        ],
        out_specs=pl.BlockSpec(memory_space=pltpu.VMEM),
        scratch_shapes=[
            pltpu.VMEM((TC_BATCH, EMB_DIM), jnp.float32),
            pltpu.VMEM((TC_BATCH, EMB_DIM), jnp.float32),
        ] + [pltpu.SemaphoreType.DMA] * 8,
    )(p1, p2, table)


def kernel(product1, product2, embedding_weight):
    p1 = product1.astype(jnp.int32)
    p2 = product2.astype(jnp.int32)
    out_sc = _sc_half(p1[:SC_BATCH], p2[:SC_BATCH], embedding_weight)
    out_tc = _tc_half(p1[SC_BATCH:], p2[SC_BATCH:], embedding_weight)
    return jnp.concatenate([out_sc, out_tc])


# speedup vs baseline: 1.2143x; 1.0230x over previous
import jax
import jax.numpy as jnp
from jax import lax
from jax.experimental import pallas as pl
from jax.experimental.pallas import tpu as pltpu
from jax.experimental.pallas import tpu_sc as plsc

EMB_ROWS = 1000000
EMB_DIM = 32
BATCH = 16384
SC_BATCH = 11264
TC_BATCH = BATCH - SC_BATCH
NUM_CORES = 2
NUM_SUBCORES = 16
LANES = 16
NUM_WORKERS = NUM_CORES * NUM_SUBCORES
ROWS_PER_WORKER = SC_BATCH // NUM_WORKERS
CHUNK = 176
NCHUNKS = ROWS_PER_WORKER // CHUNK
GROUPS = CHUNK // LANES


def _sc_body(p1_hbm, p2_hbm, table_hbm, out_hbm,
             idx1_v, idx2_v, rows1_v, rows2_v, out_v,
             sem1a, sem1b, sem2a, sem2b):
    wid = lax.axis_index("s") * NUM_CORES + lax.axis_index("c")
    base = wid * ROWS_PER_WORKER

    pltpu.sync_copy(p1_hbm.at[pl.ds(base, ROWS_PER_WORKER)], idx1_v)
    pltpu.sync_copy(p2_hbm.at[pl.ds(base, ROWS_PER_WORKER)], idx2_v)

    sems1 = (sem1a, sem1b)
    sems2 = (sem2a, sem2b)

    def start_chunk(c, buf):
        def issue(g, carry):
            iv1 = idx1_v[pl.ds(c * CHUNK + g * LANES, LANES)]
            iv2 = idx2_v[pl.ds(c * CHUNK + g * LANES, LANES)]
            for r in range(LANES):
                slot = g * LANES + r
                pltpu.async_copy(table_hbm.at[pl.ds(iv1[r], 1)],
                                 rows1_v.at[buf, pl.ds(slot, 1)], sems1[buf])
                pltpu.async_copy(table_hbm.at[pl.ds(iv2[r], 1)],
                                 rows2_v.at[buf, pl.ds(slot, 1)], sems2[buf])
            return carry
        lax.fori_loop(0, GROUPS, issue, 0)

    def wait_chunk(buf):
        def drain(r, carry):
            pltpu.make_async_copy(table_hbm.at[pl.ds(0, 1)],
                                  rows1_v.at[buf, pl.ds(0, 1)],
                                  sems1[buf]).wait()
            pltpu.make_async_copy(table_hbm.at[pl.ds(0, 1)],
                                  rows2_v.at[buf, pl.ds(0, 1)],
                                  sems2[buf]).wait()
            return carry
        lax.fori_loop(0, CHUNK, drain, 0)

    lane = lax.iota(jnp.int32, LANES)

    def shuffle(v, perm):
        return lax.gather(
            v, perm[:, None],
            lax.GatherDimensionNumbers(
                offset_dims=(), collapsed_slice_dims=(0,),
                start_index_map=(0,)),
            slice_sizes=(1,),
            mode=lax.GatherScatterMode.PROMISE_IN_BOUNDS)

    def combine(a, b, k):
        m = (lane & k) == 0
        sel_ab = jnp.where(m, a, b)
        sel_ba = jnp.where(m, b, a)
        return sel_ab + shuffle(sel_ba, lane ^ k)

    def compute_chunk(buf, out_base):
        r1 = rows1_v.at[buf]
        r2 = rows2_v.at[buf]

        def group(g, carry):
            row0 = g * LANES
            w = []
            for r in range(LANES):
                row = row0 + r
                a0 = r1[row, pl.ds(0, LANES)]
                a1 = r1[row, pl.ds(LANES, LANES)]
                b0 = r2[row, pl.ds(0, LANES)]
                b1 = r2[row, pl.ds(LANES, LANES)]
                w.append(a0 * b0 + a1 * b1)
            for k in (1, 2, 4, 8):
                w = [combine(w[2 * i], w[2 * i + 1], k)
                     for i in range(len(w) // 2)]
            acc = w[0]
            out_v[pl.ds(out_base + row0, LANES)] = 1.0 / (1.0 + jnp.exp(-acc))
            return carry

        lax.fori_loop(0, GROUPS, group, 0)

    start_chunk(0, 0)
    for c in range(NCHUNKS):
        buf = c % 2
        if c + 1 < NCHUNKS:
            start_chunk(c + 1, 1 - buf)
        wait_chunk(buf)
        compute_chunk(buf, c * CHUNK)

    pltpu.sync_copy(out_v, out_hbm.at[pl.ds(base, ROWS_PER_WORKER)])


def _sc_half(p1, p2, table):
    mesh = plsc.VectorSubcoreMesh(core_axis_name="c", subcore_axis_name="s")
    run = pl.kernel(
        _sc_body,
        mesh=mesh,
        out_type=jax.ShapeDtypeStruct((SC_BATCH,), jnp.float32),
        scratch_types=[
            pltpu.VMEM((ROWS_PER_WORKER,), jnp.int32),
            pltpu.VMEM((ROWS_PER_WORKER,), jnp.int32),
            pltpu.VMEM((2, CHUNK, EMB_DIM), jnp.float32),
            pltpu.VMEM((2, CHUNK, EMB_DIM), jnp.float32),
            pltpu.VMEM((ROWS_PER_WORKER,), jnp.float32),
            pltpu.SemaphoreType.DMA,
            pltpu.SemaphoreType.DMA,
            pltpu.SemaphoreType.DMA,
            pltpu.SemaphoreType.DMA,
        ],
    )
    return run(p1, p2, table)


def _tc_body(p1_s, p2_s, table_hbm, out_v, rows1_v, rows2_v, *sems):
    def issue(g, carry):
        for u in range(8):
            r = g * 8 + u
            i1 = p1_s[r]
            i2 = p2_s[r]
            pltpu.make_async_copy(table_hbm.at[pl.ds(i1, 1)],
                                  rows1_v.at[pl.ds(r, 1)], sems[0]).start()
            pltpu.make_async_copy(table_hbm.at[pl.ds(i2, 1)],
                                  rows2_v.at[pl.ds(r, 1)], sems[1]).start()
        return carry

    lax.fori_loop(0, TC_BATCH // 8, issue, 0)

    pltpu.make_async_copy(table_hbm.at[pl.ds(0, TC_BATCH)],
                          rows1_v, sems[0]).wait()
    pltpu.make_async_copy(table_hbm.at[pl.ds(0, TC_BATCH)],
                          rows2_v, sems[1]).wait()

    prod = rows1_v[...] * rows2_v[...]
    s = jnp.sum(prod, axis=1)
    out_v[...] = 1.0 / (1.0 + jnp.exp(-s))


def _tc_half(p1, p2, table):
    return pl.pallas_call(
        _tc_body,
        out_shape=jax.ShapeDtypeStruct((TC_BATCH,), jnp.float32),
        in_specs=[
            pl.BlockSpec(memory_space=pltpu.SMEM),
            pl.BlockSpec(memory_space=pltpu.SMEM),
            pl.BlockSpec(memory_space=pl.ANY),
        ],
        out_specs=pl.BlockSpec(memory_space=pltpu.VMEM),
        scratch_shapes=[
            pltpu.VMEM((TC_BATCH, EMB_DIM), jnp.float32),
            pltpu.VMEM((TC_BATCH, EMB_DIM), jnp.float32),
        ] + [pltpu.SemaphoreType.DMA] * 2,
    )(p1, p2, table)


def kernel(product1, product2, embedding_weight):
    p1 = product1.astype(jnp.int32)
    p2 = product2.astype(jnp.int32)
    out_sc = _sc_half(p1[:SC_BATCH], p2[:SC_BATCH], embedding_weight)
    out_tc = _tc_half(p1[SC_BATCH:], p2[SC_BATCH:], embedding_weight)
    return jnp.concatenate([out_sc, out_tc])
